# initial kernel scaffold (unmeasured)
import jax
import jax.numpy as jnp
from jax import lax
from jax.experimental import pallas as pl
from jax.experimental.pallas import tpu as pltpu

N_DEV = 8
B, SQ, D_MODEL = 2, 512, 768
HQ, DH = 8, 64
HD = HQ * DH
SKV_SHARD = 512
WIN = 128
SKV_USED = SQ + WIN
LO = SKV_SHARD
HI = SKV_USED - SKV_SHARD


def kernel(x, Wq, K_ext, V_ext, Wo):
    xb = x.astype(jnp.bfloat16)
    wqb = Wq.astype(jnp.bfloat16)
    wob = Wo.astype(jnp.bfloat16)
    kb = K_ext.reshape(B, SKV_SHARD, 64 * DH).astype(jnp.bfloat16)
    vb = V_ext.reshape(B, SKV_SHARD, 64 * DH).astype(jnp.bfloat16)

    def body(x_ref, wq_ref, k_ref, v_ref, wo_ref, out_ref,
             kbuf, vbuf, ringbuf,
             kv_send_sems, kv_recv_sems, ring_send_sems, ring_recv_sems):
        my = lax.axis_index("i")
        right = lax.rem(my + 1, N_DEV)

        @pl.when(my == 0)
        def _():
            rdmas = []
            for t in range(1, N_DEV):
                for ci, (src, dst) in enumerate(((k_ref, kbuf), (v_ref, vbuf))):
                    r = pltpu.make_async_remote_copy(
                        src_ref=src.at[:, :, pl.ds(t * HD, HD)],
                        dst_ref=dst.at[:, pl.ds(0, LO), :],
                        send_sem=kv_send_sems.at[ci, t],
                        recv_sem=kv_recv_sems.at[0, ci],
                        device_id=(t,),
                        device_id_type=pl.DeviceIdType.MESH,
                    )
                    r.start()
                    rdmas.append(r)
            kbuf[:, :LO, :] = k_ref[:, :, :HD]
            vbuf[:, :LO, :] = v_ref[:, :, :HD]
            for r in rdmas:
                r.wait_send()

        @pl.when(my == 1)
        def _():
            rdmas = []
            for t in (0, 2, 3, 4, 5, 6, 7):
                for ci, (src, dst) in enumerate(((k_ref, kbuf), (v_ref, vbuf))):
                    r = pltpu.make_async_remote_copy(
                        src_ref=src.at[:, pl.ds(0, HI), pl.ds(t * HD, HD)],
                        dst_ref=dst.at[:, pl.ds(LO, HI), :],
                        send_sem=kv_send_sems.at[ci, t],
                        recv_sem=kv_recv_sems.at[1, ci],
                        device_id=(t,),
                        device_id_type=pl.DeviceIdType.MESH,
                    )
                    r.start()
                    rdmas.append(r)
            kbuf[:, LO:, :] = k_ref[:, :HI, HD:2 * HD]
            vbuf[:, LO:, :] = v_ref[:, :HI, HD:2 * HD]
            for r in rdmas:
                r.wait_send()

        @pl.when(my != 0)
        def _():
            for ci, dst in ((0, kbuf), (1, vbuf)):
                pltpu.make_async_remote_copy(
                    src_ref=dst.at[:, pl.ds(0, LO), :],
                    dst_ref=dst.at[:, pl.ds(0, LO), :],
                    send_sem=kv_send_sems.at[ci, 0],
                    recv_sem=kv_recv_sems.at[0, ci],
                    device_id=(0,),
                    device_id_type=pl.DeviceIdType.MESH,
                ).wait_recv()

        @pl.when(my != 1)
        def _():
            for ci, dst in ((0, kbuf), (1, vbuf)):
                pltpu.make_async_remote_copy(
                    src_ref=dst.at[:, pl.ds(LO, HI), :],
                    dst_ref=dst.at[:, pl.ds(LO, HI), :],
                    send_sem=kv_send_sems.at[ci, 1],
                    recv_sem=kv_recv_sems.at[1, ci],
                    device_id=(1,),
                    device_id_type=pl.DeviceIdType.MESH,
                ).wait_recv()

        qi = lax.broadcasted_iota(jnp.int32, (SQ, SKV_USED), 0)
        ki = lax.broadcasted_iota(jnp.int32, (SQ, SKV_USED), 1)
        bias = jnp.where(jnp.abs(qi - ki) <= WIN, 0.0, -1e9).astype(jnp.float32)

        for b in range(B):
            q_b = lax.dot_general(
                x_ref[b], wq_ref[...],
                (((1,), (0,)), ((), ())),
                preferred_element_type=jnp.float32,
            ).astype(jnp.bfloat16)
            ctx_heads = []
            for h in range(HQ):
                qh = q_b[:, h * DH:(h + 1) * DH]
                kh = kbuf[b, :, h * DH:(h + 1) * DH]
                scores = lax.dot_general(
                    qh, kh, (((1,), (1,)), ((), ())),
                    preferred_element_type=jnp.float32,
                ) * 0.125 + bias
                m = jnp.max(scores, axis=-1, keepdims=True)
                w = jnp.exp(scores - m)
                w = (w / jnp.sum(w, axis=-1, keepdims=True)).astype(jnp.bfloat16)
                vh = vbuf[b, :, h * DH:(h + 1) * DH]
                ctx_heads.append(lax.dot_general(
                    w, vh, (((1,), (0,)), ((), ())),
                    preferred_element_type=jnp.float32,
                ).astype(jnp.bfloat16))
            ctx_b = jnp.concatenate(ctx_heads, axis=1)
            partial_b = lax.dot_general(
                ctx_b, wo_ref[...], (((1,), (0,)), ((), ())),
                preferred_element_type=jnp.float32,
            )
            out_ref[b] = partial_b
            ringbuf[0, b] = partial_b.astype(jnp.bfloat16)

        for h in range(N_DEV - 1):
            r = pltpu.make_async_remote_copy(
                src_ref=ringbuf.at[h],
                dst_ref=ringbuf.at[h + 1],
                send_sem=ring_send_sems.at[h],
                recv_sem=ring_recv_sems.at[h],
                device_id=(right,),
                device_id_type=pl.DeviceIdType.MESH,
            )
            r.start()
            r.wait()
            out_ref[...] = out_ref[...] + ringbuf[h + 1].astype(jnp.float32)

    return pl.pallas_call(
        body,
        out_shape=jax.ShapeDtypeStruct((B, SQ, D_MODEL), jnp.float32),
        in_specs=[pl.BlockSpec(memory_space=pltpu.VMEM)] * 5,
        out_specs=pl.BlockSpec(memory_space=pltpu.VMEM),
        scratch_shapes=[
            pltpu.VMEM((B, SKV_USED, HD), jnp.bfloat16),
            pltpu.VMEM((B, SKV_USED, HD), jnp.bfloat16),
            pltpu.VMEM((N_DEV, B, SQ, D_MODEL), jnp.bfloat16),
            pltpu.SemaphoreType.DMA((2, N_DEV)),
            pltpu.SemaphoreType.DMA((2, 2)),
            pltpu.SemaphoreType.DMA((N_DEV - 1,)),
            pltpu.SemaphoreType.DMA((N_DEV - 1,)),
        ],
        compiler_params=pltpu.CompilerParams(collective_id=0),
    )(xb, wqb, kb, vb, wob)


# baseline (device time: 272894 ns/iter reference)
import jax
import jax.numpy as jnp
from jax import lax
from jax.experimental import pallas as pl
from jax.experimental.pallas import tpu as pltpu

N_DEV = 8
B, SQ, D_MODEL = 2, 512, 768
HQ, DH = 8, 64
HD = HQ * DH
SKV_SHARD = 512
WIN = 128
SKV_USED = SQ + WIN
LO = SKV_SHARD
HI = SKV_USED - SKV_SHARD


def kernel(x, Wq, K_ext, V_ext, Wo):
    xb = x.astype(jnp.bfloat16)
    wqb = Wq.astype(jnp.bfloat16)
    wob = Wo.astype(jnp.bfloat16)
    kb = K_ext.reshape(B, SKV_SHARD, 64 * DH).astype(jnp.bfloat16)
    vb = V_ext.reshape(B, SKV_SHARD, 64 * DH).astype(jnp.bfloat16)

    def body(x_ref, wq_ref, k_ref, v_ref, wo_ref, out_ref,
             kbuf, vbuf, ringbuf,
             kv_send_sems, kv_recv_sems, ring_send_sems, ring_recv_sems):
        my = lax.axis_index("i")
        right = lax.rem(my + 1, N_DEV)

        @pl.when(my == 0)
        def _():
            rdmas = []
            for t in range(1, N_DEV):
                for ci, (src, dst) in enumerate(((k_ref, kbuf), (v_ref, vbuf))):
                    r = pltpu.make_async_remote_copy(
                        src_ref=src.at[:, :, pl.ds(t * HD, HD)],
                        dst_ref=dst.at[:, pl.ds(0, LO), :],
                        send_sem=kv_send_sems.at[ci, t],
                        recv_sem=kv_recv_sems.at[0, ci],
                        device_id=(t,),
                        device_id_type=pl.DeviceIdType.MESH,
                    )
                    r.start()
                    rdmas.append(r)
            kbuf[:, :LO, :] = k_ref[:, :, :HD]
            vbuf[:, :LO, :] = v_ref[:, :, :HD]
            for r in rdmas:
                r.wait_send()

        @pl.when(my == 1)
        def _():
            rdmas = []
            for t in (0, 2, 3, 4, 5, 6, 7):
                for ci, (src, dst) in enumerate(((k_ref, kbuf), (v_ref, vbuf))):
                    r = pltpu.make_async_remote_copy(
                        src_ref=src.at[:, pl.ds(0, HI), pl.ds(t * HD, HD)],
                        dst_ref=dst.at[:, pl.ds(LO, HI), :],
                        send_sem=kv_send_sems.at[ci, t],
                        recv_sem=kv_recv_sems.at[1, ci],
                        device_id=(t,),
                        device_id_type=pl.DeviceIdType.MESH,
                    )
                    r.start()
                    rdmas.append(r)
            kbuf[:, LO:, :] = k_ref[:, :HI, HD:2 * HD]
            vbuf[:, LO:, :] = v_ref[:, :HI, HD:2 * HD]
            for r in rdmas:
                r.wait_send()

        @pl.when(my != 0)
        def _():
            for ci, dst in ((0, kbuf), (1, vbuf)):
                pltpu.make_async_remote_copy(
                    src_ref=dst.at[:, pl.ds(0, LO), :],
                    dst_ref=dst.at[:, pl.ds(0, LO), :],
                    send_sem=kv_send_sems.at[ci, 0],
                    recv_sem=kv_recv_sems.at[0, ci],
                    device_id=(0,),
                    device_id_type=pl.DeviceIdType.MESH,
                ).wait_recv()

        @pl.when(my != 1)
        def _():
            for ci, dst in ((0, kbuf), (1, vbuf)):
                pltpu.make_async_remote_copy(
                    src_ref=dst.at[:, pl.ds(LO, HI), :],
                    dst_ref=dst.at[:, pl.ds(LO, HI), :],
                    send_sem=kv_send_sems.at[ci, 1],
                    recv_sem=kv_recv_sems.at[1, ci],
                    device_id=(1,),
                    device_id_type=pl.DeviceIdType.MESH,
                ).wait_recv()

        qi = lax.broadcasted_iota(jnp.int32, (SQ, SKV_USED), 0)
        ki = lax.broadcasted_iota(jnp.int32, (SQ, SKV_USED), 1)
        bias = jnp.where(jnp.abs(qi - ki) <= WIN, 0.0, -1e9).astype(jnp.float32)

        for b in range(B):
            q_b = lax.dot_general(
                x_ref[b], wq_ref[...],
                (((1,), (0,)), ((), ())),
                preferred_element_type=jnp.float32,
            ).astype(jnp.bfloat16)
            ctx_heads = []
            for h in range(HQ):
                qh = q_b[:, h * DH:(h + 1) * DH]
                kh = kbuf[b, :, h * DH:(h + 1) * DH]
                scores = lax.dot_general(
                    qh, kh, (((1,), (1,)), ((), ())),
                    preferred_element_type=jnp.float32,
                ) * 0.125 + bias
                m = jnp.max(scores, axis=-1, keepdims=True)
                w = jnp.exp(scores - m)
                w = (w / jnp.sum(w, axis=-1, keepdims=True)).astype(jnp.bfloat16)
                vh = vbuf[b, :, h * DH:(h + 1) * DH]
                ctx_heads.append(lax.dot_general(
                    w, vh, (((1,), (0,)), ((), ())),
                    preferred_element_type=jnp.float32,
                ).astype(jnp.bfloat16))
            ctx_b = jnp.concatenate(ctx_heads, axis=1)
            partial_b = lax.dot_general(
                ctx_b, wo_ref[...], (((1,), (0,)), ((), ())),
                preferred_element_type=jnp.float32,
            )
            out_ref[b] = partial_b
            ringbuf[0, b] = partial_b.astype(jnp.bfloat16)

        for h in range(N_DEV - 1):
            r = pltpu.make_async_remote_copy(
                src_ref=ringbuf.at[h],
                dst_ref=ringbuf.at[h + 1],
                send_sem=ring_send_sems.at[h],
                recv_sem=ring_recv_sems.at[h],
                device_id=(right,),
                device_id_type=pl.DeviceIdType.MESH,
            )
            r.start()
            r.wait()
            out_ref[...] = out_ref[...] + ringbuf[h + 1].astype(jnp.float32)

    return pl.pallas_call(
        body,
        out_shape=jax.ShapeDtypeStruct((B, SQ, D_MODEL), jnp.float32),
        in_specs=[pl.BlockSpec(memory_space=pltpu.VMEM)] * 5,
        out_specs=pl.BlockSpec(memory_space=pltpu.VMEM),
        scratch_shapes=[
            pltpu.VMEM((B, SKV_USED, HD), jnp.bfloat16),
            pltpu.VMEM((B, SKV_USED, HD), jnp.bfloat16),
            pltpu.VMEM((N_DEV, B, SQ, D_MODEL), jnp.bfloat16),
            pltpu.SemaphoreType.DMA((2, N_DEV)),
            pltpu.SemaphoreType.DMA((2, 2)),
            pltpu.SemaphoreType.DMA((N_DEV - 1,)),
            pltpu.SemaphoreType.DMA((N_DEV - 1,)),
        ],
    )(xb, wqb, kb, vb, wob)


# device time: 183738 ns/iter; 1.4852x vs baseline; 1.4852x over previous
import jax
import jax.numpy as jnp
from jax import lax
from jax.experimental import pallas as pl
from jax.experimental.pallas import tpu as pltpu

N_DEV = 8
B, SQ, D_MODEL = 2, 512, 768
HQ, DH = 8, 64
HD = HQ * DH
SKV_SHARD = 512
WIN = 128
SKV_USED = SQ + WIN
LO = SKV_SHARD
HI = SKV_USED - SKV_SHARD


def kernel(x, Wq, K_ext, V_ext, Wo):
    xb = x.astype(jnp.bfloat16)
    wqb = Wq.astype(jnp.bfloat16)
    wob = Wo.astype(jnp.bfloat16)
    kb = K_ext.reshape(B, SKV_SHARD, 64 * DH).astype(jnp.bfloat16)
    vb = V_ext.reshape(B, SKV_SHARD, 64 * DH).astype(jnp.bfloat16)

    def body(x_ref, wq_ref, k_ref, v_ref, wo_ref, out_ref,
             kbuf, vbuf, rsbuf, sendbuf, agbuf,
             kv_send_sems, kv_recv_sems,
             rs_send_sems, rs_recv_sems, ag_send_sems, ag_recv_sems):
        my = lax.axis_index("i")

        @pl.when(my == 0)
        def _():
            rdmas = []
            for ci, (src, dst) in enumerate(((k_ref, kbuf), (v_ref, vbuf))):
                for t in range(1, N_DEV):
                    r = pltpu.make_async_remote_copy(
                        src_ref=src.at[:, :, pl.ds(t * HD, HD)],
                        dst_ref=dst.at[:, pl.ds(0, LO), :],
                        send_sem=kv_send_sems.at[ci, t],
                        recv_sem=kv_recv_sems.at[0, ci],
                        device_id=(t,),
                        device_id_type=pl.DeviceIdType.MESH,
                    )
                    r.start()
                    rdmas.append(r)
            kbuf[:, :LO, :] = k_ref[:, :, :HD]
            vbuf[:, :LO, :] = v_ref[:, :, :HD]
            for r in rdmas:
                r.wait_send()

        @pl.when(my == 1)
        def _():
            rdmas = []
            for ci, (src, dst) in enumerate(((k_ref, kbuf), (v_ref, vbuf))):
                for t in (0, 2, 3, 4, 5, 6, 7):
                    r = pltpu.make_async_remote_copy(
                        src_ref=src.at[:, pl.ds(0, HI), pl.ds(t * HD, HD)],
                        dst_ref=dst.at[:, pl.ds(LO, HI), :],
                        send_sem=kv_send_sems.at[ci, t],
                        recv_sem=kv_recv_sems.at[1, ci],
                        device_id=(t,),
                        device_id_type=pl.DeviceIdType.MESH,
                    )
                    r.start()
                    rdmas.append(r)
            kbuf[:, LO:, :] = k_ref[:, :HI, HD:2 * HD]
            vbuf[:, LO:, :] = v_ref[:, :HI, HD:2 * HD]
            for r in rdmas:
                r.wait_send()

        qi = lax.broadcasted_iota(jnp.int32, (SQ, SKV_USED), 0)
        ki = lax.broadcasted_iota(jnp.int32, (SQ, SKV_USED), 1)
        bias = jnp.where(jnp.abs(qi - ki) <= WIN, 0.0, -1e9).astype(jnp.float32)
        qs = [
            lax.dot_general(
                x_ref[b], wq_ref[...],
                (((1,), (0,)), ((), ())),
                preferred_element_type=jnp.float32,
            ).astype(jnp.bfloat16)
            for b in range(B)
        ]

        @pl.when(my != 0)
        def _():
            for ci, dst in ((0, kbuf), (1, vbuf)):
                pltpu.make_async_remote_copy(
                    src_ref=dst.at[:, pl.ds(0, LO), :],
                    dst_ref=dst.at[:, pl.ds(0, LO), :],
                    send_sem=kv_send_sems.at[ci, 0],
                    recv_sem=kv_recv_sems.at[0, ci],
                    device_id=(0,),
                    device_id_type=pl.DeviceIdType.MESH,
                ).wait_recv()

        @pl.when(my != 1)
        def _():
            for ci, dst in ((0, kbuf), (1, vbuf)):
                pltpu.make_async_remote_copy(
                    src_ref=dst.at[:, pl.ds(LO, HI), :],
                    dst_ref=dst.at[:, pl.ds(LO, HI), :],
                    send_sem=kv_send_sems.at[ci, 1],
                    recv_sem=kv_recv_sems.at[1, ci],
                    device_id=(1,),
                    device_id_type=pl.DeviceIdType.MESH,
                ).wait_recv()

        for b in range(B):
            q_b = qs[b]
            ctx_heads = []
            for h in range(HQ):
                qh = q_b[:, h * DH:(h + 1) * DH]
                kh = kbuf[b, :, h * DH:(h + 1) * DH]
                scores = lax.dot_general(
                    qh, kh, (((1,), (1,)), ((), ())),
                    preferred_element_type=jnp.float32,
                ) * 0.125 + bias
                m = jnp.max(scores, axis=-1, keepdims=True)
                w = jnp.exp(scores - m)
                w = (w / jnp.sum(w, axis=-1, keepdims=True)).astype(jnp.bfloat16)
                vh = vbuf[b, :, h * DH:(h + 1) * DH]
                ctx_heads.append(lax.dot_general(
                    w, vh, (((1,), (0,)), ((), ())),
                    preferred_element_type=jnp.float32,
                ).astype(jnp.bfloat16))
            ctx_b = jnp.concatenate(ctx_heads, axis=1)
            partial_b = lax.dot_general(
                ctx_b, wo_ref[...], (((1,), (0,)), ((), ())),
                preferred_element_type=jnp.float32,
            )
            out_ref[b] = partial_b

        L = my ^ ((my >> 1) & 1)

        def partner(s):
            pL = L ^ (1 << s)
            return pL ^ ((pL >> 1) & 1)

        cur_off = my * 0
        seg_offs = {2: 0, 1: 256, 0: 384}
        for si, s in enumerate((2, 1, 0)):
            half = 64 << s
            mybit = (L >> s) & 1
            send_off = cur_off + (1 - mybit) * half
            keep_off = cur_off + mybit * half
            sendbuf[:, :half, :] = out_ref[:, pl.ds(send_off, half), :].astype(
                jnp.bfloat16)
            r = pltpu.make_async_remote_copy(
                src_ref=sendbuf.at[:, pl.ds(0, half)],
                dst_ref=rsbuf.at[:, pl.ds(seg_offs[s], half)],
                send_sem=rs_send_sems.at[si],
                recv_sem=rs_recv_sems.at[si],
                device_id=(partner(s),),
                device_id_type=pl.DeviceIdType.MESH,
            )
            r.start()
            r.wait()
            out_ref[:, pl.ds(keep_off, half), :] = (
                out_ref[:, pl.ds(keep_off, half), :]
                + rsbuf[:, pl.ds(seg_offs[s], half), :].astype(jnp.float32)
            )
            cur_off = keep_off

        own_off = cur_off
        agbuf[:, pl.ds(own_off, 64), :] = out_ref[:, pl.ds(own_off, 64), :].astype(
            jnp.bfloat16)
        for si, s in enumerate((0, 1, 2)):
            blk = 64 << s
            r = pltpu.make_async_remote_copy(
                src_ref=agbuf.at[:, pl.ds(own_off, blk)],
                dst_ref=agbuf.at[:, pl.ds(own_off, blk)],
                send_sem=ag_send_sems.at[si],
                recv_sem=ag_recv_sems.at[si],
                device_id=(partner(s),),
                device_id_type=pl.DeviceIdType.MESH,
            )
            r.start()
            r.wait()
            own_off = own_off - ((L >> s) & 1) * blk
        out_ref[...] = agbuf[...].astype(jnp.float32)

    return pl.pallas_call(
        body,
        out_shape=jax.ShapeDtypeStruct((B, SQ, D_MODEL), jnp.float32),
        in_specs=[pl.BlockSpec(memory_space=pltpu.VMEM)] * 5,
        out_specs=pl.BlockSpec(memory_space=pltpu.VMEM),
        scratch_shapes=[
            pltpu.VMEM((B, SKV_USED, HD), jnp.bfloat16),
            pltpu.VMEM((B, SKV_USED, HD), jnp.bfloat16),
            pltpu.VMEM((B, 448, D_MODEL), jnp.bfloat16),
            pltpu.VMEM((B, 256, D_MODEL), jnp.bfloat16),
            pltpu.VMEM((B, SQ, D_MODEL), jnp.bfloat16),
            pltpu.SemaphoreType.DMA((2, N_DEV)),
            pltpu.SemaphoreType.DMA((2, 2)),
            pltpu.SemaphoreType.DMA((3,)),
            pltpu.SemaphoreType.DMA((3,)),
            pltpu.SemaphoreType.DMA((3,)),
            pltpu.SemaphoreType.DMA((3,)),
        ],
    )(xb, wqb, kb, vb, wob)


# device time: 183106 ns/iter; 1.4904x vs baseline; 1.0035x over previous
import jax
import jax.numpy as jnp
from jax import lax
from jax.experimental import pallas as pl
from jax.experimental.pallas import tpu as pltpu

N_DEV = 8
B, SQ, D_MODEL = 2, 512, 768
HQ, DH = 8, 64
HD = HQ * DH
SKV_SHARD = 512
WIN = 128
SKV_USED = SQ + WIN
LO = SKV_SHARD
HI = SKV_USED - SKV_SHARD


def kernel(x, Wq, K_ext, V_ext, Wo):
    xb = x.astype(jnp.bfloat16)
    wqb = Wq.astype(jnp.bfloat16)
    wob = Wo.astype(jnp.bfloat16)
    kb = K_ext.reshape(B, SKV_SHARD, 64 * DH).astype(jnp.bfloat16)
    vb = V_ext.reshape(B, SKV_SHARD, 64 * DH).astype(jnp.bfloat16)

    def body(x_ref, wq_ref, k_ref, v_ref, wo_ref, out_ref,
             kbuf, vbuf, rsbuf, sendbuf, agbuf,
             kv_send_sems, kv_recv_sems,
             rs_send_sems, rs_recv_sems, ag_send_sems, ag_recv_sems):
        my = lax.axis_index("i")

        def dev0_flows(ci, src, dst):
            return [
                pltpu.make_async_remote_copy(
                    src_ref=src.at[:, :, pl.ds(t * HD, HD)],
                    dst_ref=dst.at[:, pl.ds(0, LO), :],
                    send_sem=kv_send_sems.at[ci, t],
                    recv_sem=kv_recv_sems.at[0, ci],
                    device_id=(t,),
                    device_id_type=pl.DeviceIdType.MESH,
                )
                for t in range(1, N_DEV)
            ]

        def dev1_flows(ci, src, dst):
            return [
                pltpu.make_async_remote_copy(
                    src_ref=src.at[:, pl.ds(0, HI), pl.ds(t * HD, HD)],
                    dst_ref=dst.at[:, pl.ds(LO, HI), :],
                    send_sem=kv_send_sems.at[ci, t],
                    recv_sem=kv_recv_sems.at[1, ci],
                    device_id=(t,),
                    device_id_type=pl.DeviceIdType.MESH,
                )
                for t in (0, 2, 3, 4, 5, 6, 7)
            ]

        def wait_kv_recv(src_rank, ci, dst):
            region = pl.ds(0, LO) if src_rank == 0 else pl.ds(LO, HI)
            pltpu.make_async_remote_copy(
                src_ref=dst.at[:, region, :],
                dst_ref=dst.at[:, region, :],
                send_sem=kv_send_sems.at[ci, src_rank],
                recv_sem=kv_recv_sems.at[src_rank, ci],
                device_id=(src_rank,),
                device_id_type=pl.DeviceIdType.MESH,
            ).wait_recv()

        @pl.when(my == 0)
        def _():
            for r in dev0_flows(0, k_ref, kbuf):
                r.start()
            kbuf[:, :LO, :] = k_ref[:, :, :HD]

        @pl.when(my == 1)
        def _():
            for ci, (src, dst) in enumerate(((k_ref, kbuf), (v_ref, vbuf))):
                for r in dev1_flows(ci, src, dst):
                    r.start()
            kbuf[:, LO:, :] = k_ref[:, :HI, HD:2 * HD]
            vbuf[:, LO:, :] = v_ref[:, :HI, HD:2 * HD]

        qi = lax.broadcasted_iota(jnp.int32, (SQ, SKV_USED), 0)
        ki = lax.broadcasted_iota(jnp.int32, (SQ, SKV_USED), 1)
        bias = jnp.where(jnp.abs(qi - ki) <= WIN, 0.0, -1e9).astype(jnp.float32)
        qs = [
            lax.dot_general(
                x_ref[b], wq_ref[...],
                (((1,), (0,)), ((), ())),
                preferred_element_type=jnp.float32,
            ).astype(jnp.bfloat16)
            for b in range(B)
        ]

        @pl.when(my == 0)
        def _():
            for r in dev0_flows(0, k_ref, kbuf):
                r.wait_send()
            for r in dev0_flows(1, v_ref, vbuf):
                r.start()
            vbuf[:, :LO, :] = v_ref[:, :, :HD]

        @pl.when(my != 0)
        def _():
            wait_kv_recv(0, 0, kbuf)

        @pl.when(my != 1)
        def _():
            wait_kv_recv(1, 0, kbuf)

        ws = []
        for b in range(B):
            for h in range(HQ):
                qh = qs[b][:, h * DH:(h + 1) * DH]
                kh = kbuf[b, :, h * DH:(h + 1) * DH]
                scores = lax.dot_general(
                    qh, kh, (((1,), (1,)), ((), ())),
                    preferred_element_type=jnp.float32,
                ) * 0.125 + bias
                m = jnp.max(scores, axis=-1, keepdims=True)
                w = jnp.exp(scores - m)
                ws.append(
                    (w / jnp.sum(w, axis=-1, keepdims=True)).astype(jnp.bfloat16))

        @pl.when(my == 0)
        def _():
            for r in dev0_flows(1, v_ref, vbuf):
                r.wait_send()

        @pl.when(my == 1)
        def _():
            for ci, (src, dst) in enumerate(((k_ref, kbuf), (v_ref, vbuf))):
                for r in dev1_flows(ci, src, dst):
                    r.wait_send()

        @pl.when(my != 0)
        def _():
            wait_kv_recv(0, 1, vbuf)

        @pl.when(my != 1)
        def _():
            wait_kv_recv(1, 1, vbuf)

        for b in range(B):
            ctx_heads = []
            for h in range(HQ):
                vh = vbuf[b, :, h * DH:(h + 1) * DH]
                ctx_heads.append(lax.dot_general(
                    ws[b * HQ + h], vh, (((1,), (0,)), ((), ())),
                    preferred_element_type=jnp.float32,
                ).astype(jnp.bfloat16))
            ctx_b = jnp.concatenate(ctx_heads, axis=1)
            partial_b = lax.dot_general(
                ctx_b, wo_ref[...], (((1,), (0,)), ((), ())),
                preferred_element_type=jnp.float32,
            )
            out_ref[b] = partial_b

        L = my ^ ((my >> 1) & 1)

        def partner(s):
            pL = L ^ (1 << s)
            return pL ^ ((pL >> 1) & 1)

        cur_off = my * 0
        seg_offs = {2: 0, 1: 256, 0: 384}
        for si, s in enumerate((2, 1, 0)):
            half = 64 << s
            mybit = (L >> s) & 1
            send_off = cur_off + (1 - mybit) * half
            keep_off = cur_off + mybit * half
            sendbuf[:, :half, :] = out_ref[:, pl.ds(send_off, half), :].astype(
                jnp.bfloat16)
            r = pltpu.make_async_remote_copy(
                src_ref=sendbuf.at[:, pl.ds(0, half)],
                dst_ref=rsbuf.at[:, pl.ds(seg_offs[s], half)],
                send_sem=rs_send_sems.at[si],
                recv_sem=rs_recv_sems.at[si],
                device_id=(partner(s),),
                device_id_type=pl.DeviceIdType.MESH,
            )
            r.start()
            r.wait()
            out_ref[:, pl.ds(keep_off, half), :] = (
                out_ref[:, pl.ds(keep_off, half), :]
                + rsbuf[:, pl.ds(seg_offs[s], half), :].astype(jnp.float32)
            )
            cur_off = keep_off

        own_off = cur_off
        agbuf[:, pl.ds(own_off, 64), :] = out_ref[:, pl.ds(own_off, 64), :].astype(
            jnp.bfloat16)
        for si, s in enumerate((0, 1, 2)):
            blk = 64 << s
            r = pltpu.make_async_remote_copy(
                src_ref=agbuf.at[:, pl.ds(own_off, blk)],
                dst_ref=agbuf.at[:, pl.ds(own_off, blk)],
                send_sem=ag_send_sems.at[si],
                recv_sem=ag_recv_sems.at[si],
                device_id=(partner(s),),
                device_id_type=pl.DeviceIdType.MESH,
            )
            r.start()
            r.wait()
            own_off = own_off - ((L >> s) & 1) * blk
        out_ref[...] = agbuf[...].astype(jnp.float32)

    return pl.pallas_call(
        body,
        out_shape=jax.ShapeDtypeStruct((B, SQ, D_MODEL), jnp.float32),
        in_specs=[pl.BlockSpec(memory_space=pltpu.VMEM)] * 5,
        out_specs=pl.BlockSpec(memory_space=pltpu.VMEM),
        scratch_shapes=[
            pltpu.VMEM((B, SKV_USED, HD), jnp.bfloat16),
            pltpu.VMEM((B, SKV_USED, HD), jnp.bfloat16),
            pltpu.VMEM((B, 448, D_MODEL), jnp.bfloat16),
            pltpu.VMEM((B, 256, D_MODEL), jnp.bfloat16),
            pltpu.VMEM((B, SQ, D_MODEL), jnp.bfloat16),
            pltpu.SemaphoreType.DMA((2, N_DEV)),
            pltpu.SemaphoreType.DMA((2, 2)),
            pltpu.SemaphoreType.DMA((3,)),
            pltpu.SemaphoreType.DMA((3,)),
            pltpu.SemaphoreType.DMA((3,)),
            pltpu.SemaphoreType.DMA((3,)),
        ],
        compiler_params=pltpu.CompilerParams(
            vmem_limit_bytes=100 * 1024 * 1024),
    )(xb, wqb, kb, vb, wob)


# device time: 99782 ns/iter; 2.7349x vs baseline; 1.8351x over previous
import os

import jax
import jax.numpy as jnp
from jax import lax
from jax.experimental import pallas as pl
from jax.experimental.pallas import tpu as pltpu

_PROBE = os.environ.get("KERNEL_PROBE", "")

N_DEV = 8
B, SQ, D_MODEL = 2, 512, 768
HQ, DH = 8, 64
HD = HQ * DH
SKV_SHARD = 512
WIN = 128
SKV_USED = SQ + WIN
LO = SKV_SHARD
HI = SKV_USED - SKV_SHARD


def kernel(x, Wq, K_ext, V_ext, Wo):
    xb = x.astype(jnp.bfloat16)
    wqb = Wq.astype(jnp.bfloat16)
    wob = Wo.astype(jnp.bfloat16)
    kb = K_ext.reshape(B, SKV_SHARD, 64 * DH).astype(jnp.bfloat16)
    vb = V_ext.reshape(B, SKV_SHARD, 64 * DH).astype(jnp.bfloat16)

    def body(x_ref, wq_ref, k_ref, v_ref, wo_ref, out_ref,
             kbuf, vbuf, rsbuf, sendbuf, agbuf,
             kv_send_sems, kv_recv_sems,
             rs_send_sems, rs_recv_sems, ag_send_sems, ag_recv_sems):
        my = lax.axis_index("i")

        def dev0_flows(ci, src, dst):
            return [
                pltpu.make_async_remote_copy(
                    src_ref=src.at[:, :, pl.ds(t * HD, HD)],
                    dst_ref=dst.at[:, pl.ds(0, LO), :],
                    send_sem=kv_send_sems.at[ci, t],
                    recv_sem=kv_recv_sems.at[0, ci],
                    device_id=(t,),
                    device_id_type=pl.DeviceIdType.MESH,
                )
                for t in range(1, N_DEV)
            ]

        def dev1_flows(ci, src, dst):
            return [
                pltpu.make_async_remote_copy(
                    src_ref=src.at[:, pl.ds(0, HI), pl.ds(t * HD, HD)],
                    dst_ref=dst.at[:, pl.ds(LO, HI), :],
                    send_sem=kv_send_sems.at[ci, t],
                    recv_sem=kv_recv_sems.at[1, ci],
                    device_id=(t,),
                    device_id_type=pl.DeviceIdType.MESH,
                )
                for t in (0, 2, 3, 4, 5, 6, 7)
            ]

        def wait_kv_recv(src_rank, ci, dst):
            region = pl.ds(0, LO) if src_rank == 0 else pl.ds(LO, HI)
            pltpu.make_async_remote_copy(
                src_ref=dst.at[:, region, :],
                dst_ref=dst.at[:, region, :],
                send_sem=kv_send_sems.at[ci, src_rank],
                recv_sem=kv_recv_sems.at[src_rank, ci],
                device_id=(src_rank,),
                device_id_type=pl.DeviceIdType.MESH,
            ).wait_recv()

        do_scatter = _PROBE != "no_scatter"

        if do_scatter:
            @pl.when(my == 0)
            def _():
                for r in dev0_flows(0, k_ref, kbuf):
                    r.start()
                kbuf[:, :LO, :] = k_ref[:, :, :HD]

            @pl.when(my == 1)
            def _():
                for ci, (src, dst) in enumerate(((k_ref, kbuf), (v_ref, vbuf))):
                    for r in dev1_flows(ci, src, dst):
                        r.start()
                kbuf[:, LO:, :] = k_ref[:, :HI, HD:2 * HD]
                vbuf[:, LO:, :] = v_ref[:, :HI, HD:2 * HD]

        qi = lax.broadcasted_iota(jnp.int32, (SQ, SKV_USED), 0)
        ki = lax.broadcasted_iota(jnp.int32, (SQ, SKV_USED), 1)
        bias = jnp.where(jnp.abs(qi - ki) <= WIN, 0.0, -1e9).astype(jnp.float32)
        qs = [
            lax.dot_general(
                x_ref[b], wq_ref[...],
                (((1,), (0,)), ((), ())),
                preferred_element_type=jnp.float32,
            ).astype(jnp.bfloat16)
            for b in range(B)
        ]

        if do_scatter:
            @pl.when(my == 0)
            def _():
                for r in dev0_flows(0, k_ref, kbuf):
                    r.wait_send()
                for r in dev0_flows(1, v_ref, vbuf):
                    r.start()
                vbuf[:, :LO, :] = v_ref[:, :, :HD]

            @pl.when(my != 0)
            def _():
                wait_kv_recv(0, 0, kbuf)

            @pl.when(my != 1)
            def _():
                wait_kv_recv(1, 0, kbuf)

        ws = []
        for b in range(B):
            for h in range(HQ):
                qh = qs[b][:, h * DH:(h + 1) * DH]
                kh = kbuf[b, :, h * DH:(h + 1) * DH]
                scores = lax.dot_general(
                    qh, kh, (((1,), (1,)), ((), ())),
                    preferred_element_type=jnp.float32,
                ) * 0.125 + bias
                m = jnp.max(scores, axis=-1, keepdims=True)
                w = jnp.exp(scores - m)
                ws.append(
                    (w / jnp.sum(w, axis=-1, keepdims=True)).astype(jnp.bfloat16))

        if do_scatter:
            @pl.when(my == 0)
            def _():
                for r in dev0_flows(1, v_ref, vbuf):
                    r.wait_send()

            @pl.when(my == 1)
            def _():
                for ci, (src, dst) in enumerate(((k_ref, kbuf), (v_ref, vbuf))):
                    for r in dev1_flows(ci, src, dst):
                        r.wait_send()

            @pl.when(my != 0)
            def _():
                wait_kv_recv(0, 1, vbuf)

            @pl.when(my != 1)
            def _():
                wait_kv_recv(1, 1, vbuf)

        for b in range(B):
            ctx_heads = []
            for h in range(HQ):
                vh = vbuf[b, :, h * DH:(h + 1) * DH]
                ctx_heads.append(lax.dot_general(
                    ws[b * HQ + h], vh, (((1,), (0,)), ((), ())),
                    preferred_element_type=jnp.float32,
                ).astype(jnp.bfloat16))
            ctx_b = jnp.concatenate(ctx_heads, axis=1)
            partial_b = lax.dot_general(
                ctx_b, wo_ref[...], (((1,), (0,)), ((), ())),
                preferred_element_type=jnp.float32,
            )
            out_ref[b] = partial_b

        if _PROBE == "no_allreduce":
            return

        L = my ^ ((my >> 1) & 1)

        def partner(s):
            pL = L ^ (1 << s)
            return pL ^ ((pL >> 1) & 1)

        cur_off = my * 0
        seg_offs = {2: 0, 1: 256, 0: 384}
        for si, s in enumerate((2, 1, 0)):
            half = 64 << s
            mybit = (L >> s) & 1
            send_off = cur_off + (1 - mybit) * half
            keep_off = cur_off + mybit * half
            sendbuf[:, :half, :] = out_ref[:, pl.ds(send_off, half), :].astype(
                jnp.bfloat16)
            r = pltpu.make_async_remote_copy(
                src_ref=sendbuf.at[:, pl.ds(0, half)],
                dst_ref=rsbuf.at[:, pl.ds(seg_offs[s], half)],
                send_sem=rs_send_sems.at[si],
                recv_sem=rs_recv_sems.at[si],
                device_id=(partner(s),),
                device_id_type=pl.DeviceIdType.MESH,
            )
            r.start()
            r.wait()
            out_ref[:, pl.ds(keep_off, half), :] = (
                out_ref[:, pl.ds(keep_off, half), :]
                + rsbuf[:, pl.ds(seg_offs[s], half), :].astype(jnp.float32)
            )
            cur_off = keep_off

        own_off = cur_off
        agbuf[:, pl.ds(own_off, 64), :] = out_ref[:, pl.ds(own_off, 64), :].astype(
            jnp.bfloat16)
        for si, s in enumerate((0, 1, 2)):
            blk = 64 << s
            r = pltpu.make_async_remote_copy(
                src_ref=agbuf.at[:, pl.ds(own_off, blk)],
                dst_ref=agbuf.at[:, pl.ds(own_off, blk)],
                send_sem=ag_send_sems.at[si],
                recv_sem=ag_recv_sems.at[si],
                device_id=(partner(s),),
                device_id_type=pl.DeviceIdType.MESH,
            )
            r.start()
            r.wait()
            own_off = own_off - ((L >> s) & 1) * blk
        out_ref[...] = agbuf[...].astype(jnp.float32)

    return pl.pallas_call(
        body,
        out_shape=jax.ShapeDtypeStruct((B, SQ, D_MODEL), jnp.float32),
        in_specs=[pl.BlockSpec(memory_space=pltpu.VMEM)] * 5,
        out_specs=pl.BlockSpec(memory_space=pltpu.VMEM),
        scratch_shapes=[
            pltpu.VMEM((B, SKV_USED, HD), jnp.bfloat16),
            pltpu.VMEM((B, SKV_USED, HD), jnp.bfloat16),
            pltpu.VMEM((B, 448, D_MODEL), jnp.bfloat16),
            pltpu.VMEM((B, 256, D_MODEL), jnp.bfloat16),
            pltpu.VMEM((B, SQ, D_MODEL), jnp.bfloat16),
            pltpu.SemaphoreType.DMA((2, N_DEV)),
            pltpu.SemaphoreType.DMA((2, 2)),
            pltpu.SemaphoreType.DMA((3,)),
            pltpu.SemaphoreType.DMA((3,)),
            pltpu.SemaphoreType.DMA((3,)),
            pltpu.SemaphoreType.DMA((3,)),
        ],
        compiler_params=pltpu.CompilerParams(
            vmem_limit_bytes=100 * 1024 * 1024),
    )(xb, wqb, kb, vb, wob)
